# baseline (device time: 37960 ns/iter reference)
import jax
import jax.numpy as jnp
from jax import lax
from jax.experimental import pallas as pl
from jax.experimental.pallas import tpu as pltpu

N_LAYERS = 3


def kernel(x, Win0, Wout0, Win1, Wout1, Win2, Wout2):
    b, d_half = x.shape
    h_dim = Win0.shape[1]

    def body(x_ref, w0, o0, w1, o1, w2, o2, out_ref,
             win_buf, wout_buf, hbuf,
             stage_h, land_h, stage_o, land_o,
             win_sem, wout_sem,
             send_h, recv_h, send_o, recv_o):
        my_x = lax.axis_index("x")
        my_y = lax.axis_index("y")
        y_peer = (my_x, 1 - my_y)
        x_peer = (1 - my_x, my_y)

        wins = (w0, w1, w2)
        wouts = (o0, o1, o2)

        def start_weight_copies(l):
            s = l % 2
            pltpu.make_async_copy(wins[l], win_buf.at[s], win_sem.at[s]).start()
            pltpu.make_async_copy(
                wouts[l], wout_buf.at[s], wout_sem.at[s]
            ).start()

        start_weight_copies(0)
        start_weight_copies(1)

        barrier = pltpu.get_barrier_semaphore()
        for peer in (y_peer, x_peer):
            pl.semaphore_signal(barrier, inc=1, device_id=peer,
                                device_id_type=pl.DeviceIdType.MESH)
        pl.semaphore_wait(barrier, 2)

        x_cur = x_ref[...]
        for l in range(N_LAYERS):
            s = l % 2
            pltpu.make_async_copy(wins[l], win_buf.at[s], win_sem.at[s]).wait()
            hbuf[...] = jnp.dot(x_cur, win_buf[s],
                                preferred_element_type=jnp.float32)
            stage_h[l] = hbuf[...].astype(jnp.bfloat16)
            rdma_h = pltpu.make_async_remote_copy(
                src_ref=stage_h.at[l],
                dst_ref=land_h.at[l],
                send_sem=send_h.at[l],
                recv_sem=recv_h.at[l],
                device_id=y_peer,
                device_id_type=pl.DeviceIdType.MESH,
            )
            rdma_h.start()
            pltpu.make_async_copy(
                wouts[l], wout_buf.at[s], wout_sem.at[s]
            ).wait()
            rdma_h.wait()
            h = jnp.maximum(hbuf[...] + land_h[l].astype(jnp.float32), 0.0)

            obuf = jnp.dot(h, wout_buf[s], preferred_element_type=jnp.float32)
            stage_o[l] = obuf.astype(jnp.bfloat16)
            rdma_o = pltpu.make_async_remote_copy(
                src_ref=stage_o.at[l],
                dst_ref=land_o.at[l],
                send_sem=send_o.at[l],
                recv_sem=recv_o.at[l],
                device_id=x_peer,
                device_id_type=pl.DeviceIdType.MESH,
            )
            rdma_o.start()
            if l == 0:
                start_weight_copies(2)
            rdma_o.wait()
            x_cur = obuf + land_o[l].astype(jnp.float32)

        out_ref[...] = x_cur

    return pl.pallas_call(
        body,
        out_shape=jax.ShapeDtypeStruct((b, d_half), jnp.float32),
        in_specs=[pl.BlockSpec(memory_space=pltpu.VMEM)]
        + [pl.BlockSpec(memory_space=pltpu.MemorySpace.HBM)] * 6,
        out_specs=pl.BlockSpec(memory_space=pltpu.VMEM),
        scratch_shapes=[
            pltpu.VMEM((2, d_half, h_dim), jnp.float32),
            pltpu.VMEM((2, h_dim, d_half), jnp.float32),
            pltpu.VMEM((b, h_dim), jnp.float32),
            pltpu.VMEM((N_LAYERS, b, h_dim), jnp.bfloat16),
            pltpu.VMEM((N_LAYERS, b, h_dim), jnp.bfloat16),
            pltpu.VMEM((N_LAYERS, b, d_half), jnp.bfloat16),
            pltpu.VMEM((N_LAYERS, b, d_half), jnp.bfloat16),
            pltpu.SemaphoreType.DMA((2,)),
            pltpu.SemaphoreType.DMA((2,)),
            pltpu.SemaphoreType.DMA((N_LAYERS,)),
            pltpu.SemaphoreType.DMA((N_LAYERS,)),
            pltpu.SemaphoreType.DMA((N_LAYERS,)),
            pltpu.SemaphoreType.DMA((N_LAYERS,)),
        ],
        compiler_params=pltpu.CompilerParams(
            collective_id=0,
            vmem_limit_bytes=100 * 1024 * 1024,
        ),
    )(x, Win0, Wout0, Win1, Wout1, Win2, Wout2)


# device time: 35564 ns/iter; 1.0674x vs baseline; 1.0674x over previous
import jax
import jax.numpy as jnp
from jax import lax
from jax.experimental import pallas as pl
from jax.experimental.pallas import tpu as pltpu

N_LAYERS = 3
N_CHUNKS = 2


def kernel(x, Win0, Wout0, Win1, Wout1, Win2, Wout2):
    b, d_half = x.shape
    h_dim = Win0.shape[1]
    h_chunk = h_dim // N_CHUNKS

    def body(x_ref, w0, o0, w1, o1, w2, o2, out_ref,
             win_buf, wout_buf, hbuf,
             stage_h, land_h, stage_o, land_o,
             win_sem, wout_sem,
             send_h, recv_h, send_o, recv_o):
        my_x = lax.axis_index("x")
        my_y = lax.axis_index("y")
        y_peer = (my_x, 1 - my_y)
        x_peer = (1 - my_x, my_y)

        wins = (w0, w1, w2)
        wouts = (o0, o1, o2)

        def win_chunk_copy(l, c):
            s = l % 2
            cs = pl.ds(c * h_chunk, h_chunk)
            return pltpu.make_async_copy(
                wins[l].at[:, cs], win_buf.at[s, :, cs], win_sem.at[s, c]
            )

        def wout_copy(l):
            s = l % 2
            return pltpu.make_async_copy(
                wouts[l], wout_buf.at[s], wout_sem.at[s]
            )

        def start_weight_copies(l):
            for c in range(N_CHUNKS):
                win_chunk_copy(l, c).start()
            wout_copy(l).start()

        start_weight_copies(0)
        start_weight_copies(1)

        barrier = pltpu.get_barrier_semaphore()
        for peer in (y_peer, x_peer):
            pl.semaphore_signal(barrier, inc=1, device_id=peer,
                                device_id_type=pl.DeviceIdType.MESH)
        pl.semaphore_wait(barrier, 2)

        x_cur = x_ref[...]
        for l in range(N_LAYERS):
            s = l % 2
            rdmas_h = []
            for c in range(N_CHUNKS):
                cs = pl.ds(c * h_chunk, h_chunk)
                win_chunk_copy(l, c).wait()
                hbuf[:, cs] = jnp.dot(x_cur, win_buf[s, :, cs],
                                      preferred_element_type=jnp.float32)
                stage_h[l, c] = hbuf[:, cs].astype(jnp.bfloat16)
                rdma = pltpu.make_async_remote_copy(
                    src_ref=stage_h.at[l, c],
                    dst_ref=land_h.at[l, c],
                    send_sem=send_h.at[l, c],
                    recv_sem=recv_h.at[l, c],
                    device_id=y_peer,
                    device_id_type=pl.DeviceIdType.MESH,
                )
                rdma.start()
                rdmas_h.append(rdma)

            wout_copy(l).wait()

            o_acc = None
            for c in range(N_CHUNKS):
                cs = pl.ds(c * h_chunk, h_chunk)
                rdmas_h[c].wait()
                h_c = jnp.maximum(
                    hbuf[:, cs] + land_h[l, c].astype(jnp.float32), 0.0
                )
                part = jnp.dot(h_c, wout_buf[s, cs, :],
                               preferred_element_type=jnp.float32)
                o_acc = part if o_acc is None else o_acc + part

            stage_o[l] = o_acc.astype(jnp.bfloat16)
            rdma_o = pltpu.make_async_remote_copy(
                src_ref=stage_o.at[l],
                dst_ref=land_o.at[l],
                send_sem=send_o.at[l],
                recv_sem=recv_o.at[l],
                device_id=x_peer,
                device_id_type=pl.DeviceIdType.MESH,
            )
            rdma_o.start()
            if l == 0:
                start_weight_copies(2)
            rdma_o.wait()
            x_cur = o_acc + land_o[l].astype(jnp.float32)

        out_ref[...] = x_cur

    return pl.pallas_call(
        body,
        out_shape=jax.ShapeDtypeStruct((b, d_half), jnp.float32),
        in_specs=[pl.BlockSpec(memory_space=pltpu.VMEM)]
        + [pl.BlockSpec(memory_space=pltpu.MemorySpace.HBM)] * 6,
        out_specs=pl.BlockSpec(memory_space=pltpu.VMEM),
        scratch_shapes=[
            pltpu.VMEM((2, d_half, h_dim), jnp.float32),
            pltpu.VMEM((2, h_dim, d_half), jnp.float32),
            pltpu.VMEM((b, h_dim), jnp.float32),
            pltpu.VMEM((N_LAYERS, N_CHUNKS, b, h_chunk), jnp.bfloat16),
            pltpu.VMEM((N_LAYERS, N_CHUNKS, b, h_chunk), jnp.bfloat16),
            pltpu.VMEM((N_LAYERS, b, d_half), jnp.bfloat16),
            pltpu.VMEM((N_LAYERS, b, d_half), jnp.bfloat16),
            pltpu.SemaphoreType.DMA((2, N_CHUNKS)),
            pltpu.SemaphoreType.DMA((2,)),
            pltpu.SemaphoreType.DMA((N_LAYERS, N_CHUNKS)),
            pltpu.SemaphoreType.DMA((N_LAYERS, N_CHUNKS)),
            pltpu.SemaphoreType.DMA((N_LAYERS,)),
            pltpu.SemaphoreType.DMA((N_LAYERS,)),
        ],
        compiler_params=pltpu.CompilerParams(
            collective_id=0,
            vmem_limit_bytes=100 * 1024 * 1024,
        ),
    )(x, Win0, Wout0, Win1, Wout1, Win2, Wout2)


# device time: 33997 ns/iter; 1.1166x vs baseline; 1.0461x over previous
import jax
import jax.numpy as jnp
from jax import lax
from jax.experimental import pallas as pl
from jax.experimental.pallas import tpu as pltpu

N_LAYERS = 3
N_CHUNKS = 4


def kernel(x, Win0, Wout0, Win1, Wout1, Win2, Wout2):
    b, d_half = x.shape
    h_dim = Win0.shape[1]
    h_chunk = h_dim // N_CHUNKS

    def body(x_ref, w0, o0, w1, o1, w2, o2, out_ref,
             win_buf, wout_buf, hbuf, macc,
             stage_h, land_h, stage_o, land_o,
             win_sem, wout_sem,
             send_h, recv_h, send_o, recv_o):
        my_x = lax.axis_index("x")
        my_y = lax.axis_index("y")
        y_peer = (my_x, 1 - my_y)
        x_peer = (1 - my_x, my_y)

        wins = (w0, w1, w2)
        wouts = (o0, o1, o2)

        def win_chunk_copy(l, c):
            s = l % 2
            cs = pl.ds(c * h_chunk, h_chunk)
            return pltpu.make_async_copy(
                wins[l].at[:, cs], win_buf.at[s, :, cs], win_sem.at[s, c]
            )

        def wout_copy(l):
            s = l % 2
            return pltpu.make_async_copy(
                wouts[l], wout_buf.at[s], wout_sem.at[s]
            )

        def start_weight_copies(l):
            for c in range(N_CHUNKS):
                win_chunk_copy(l, c).start()
            wout_copy(l).start()

        start_weight_copies(0)
        start_weight_copies(1)

        barrier = pltpu.get_barrier_semaphore()
        for peer in (y_peer, x_peer):
            pl.semaphore_signal(barrier, inc=1, device_id=peer,
                                device_id_type=pl.DeviceIdType.MESH)
        pl.semaphore_wait(barrier, 2)

        x_cur = x_ref[...]
        for l in range(N_LAYERS):
            s = l % 2
            rdmas_h = []
            for c in range(N_CHUNKS):
                cs = pl.ds(c * h_chunk, h_chunk)
                if l == 0:
                    win_chunk_copy(l, c).wait()
                    hbuf[:, cs] = jnp.dot(
                        x_cur, win_buf[s, :, cs],
                        preferred_element_type=jnp.float32,
                    )
                else:
                    hbuf[:, cs] = macc[:, cs] + jnp.dot(
                        land_o[l - 1].astype(jnp.float32),
                        win_buf[s, :, cs],
                        preferred_element_type=jnp.float32,
                    )
                stage_h[l, c] = hbuf[:, cs].astype(jnp.bfloat16)
                rdma = pltpu.make_async_remote_copy(
                    src_ref=stage_h.at[l, c],
                    dst_ref=land_h.at[l, c],
                    send_sem=send_h.at[l, c],
                    recv_sem=recv_h.at[l, c],
                    device_id=y_peer,
                    device_id_type=pl.DeviceIdType.MESH,
                )
                rdma.start()
                rdmas_h.append(rdma)

            wout_copy(l).wait()

            o_acc = None
            for c in range(N_CHUNKS):
                cs = pl.ds(c * h_chunk, h_chunk)
                rdmas_h[c].wait()
                h_c = jnp.maximum(
                    hbuf[:, cs] + land_h[l, c].astype(jnp.float32), 0.0
                )
                part = jnp.dot(h_c, wout_buf[s, cs, :],
                               preferred_element_type=jnp.float32)
                o_acc = part if o_acc is None else o_acc + part

            stage_o[l] = o_acc.astype(jnp.bfloat16)
            rdma_o = pltpu.make_async_remote_copy(
                src_ref=stage_o.at[l],
                dst_ref=land_o.at[l],
                send_sem=send_o.at[l],
                recv_sem=recv_o.at[l],
                device_id=x_peer,
                device_id_type=pl.DeviceIdType.MESH,
            )
            rdma_o.start()
            if l == 0:
                start_weight_copies(2)
            if l < N_LAYERS - 1:
                s_next = (l + 1) % 2
                for c in range(N_CHUNKS):
                    cs = pl.ds(c * h_chunk, h_chunk)
                    win_chunk_copy(l + 1, c).wait()
                    macc[:, cs] = jnp.dot(
                        o_acc, win_buf[s_next, :, cs],
                        preferred_element_type=jnp.float32,
                    )
                rdma_o.wait()
            else:
                rdma_o.wait()
                out_ref[...] = o_acc + land_o[l].astype(jnp.float32)

        return

    return pl.pallas_call(
        body,
        out_shape=jax.ShapeDtypeStruct((b, d_half), jnp.float32),
        in_specs=[pl.BlockSpec(memory_space=pltpu.VMEM)]
        + [pl.BlockSpec(memory_space=pltpu.MemorySpace.HBM)] * 6,
        out_specs=pl.BlockSpec(memory_space=pltpu.VMEM),
        scratch_shapes=[
            pltpu.VMEM((2, d_half, h_dim), jnp.float32),
            pltpu.VMEM((2, h_dim, d_half), jnp.float32),
            pltpu.VMEM((b, h_dim), jnp.float32),
            pltpu.VMEM((b, h_dim), jnp.float32),
            pltpu.VMEM((N_LAYERS, N_CHUNKS, b, h_chunk), jnp.bfloat16),
            pltpu.VMEM((N_LAYERS, N_CHUNKS, b, h_chunk), jnp.bfloat16),
            pltpu.VMEM((N_LAYERS, b, d_half), jnp.bfloat16),
            pltpu.VMEM((N_LAYERS, b, d_half), jnp.bfloat16),
            pltpu.SemaphoreType.DMA((2, N_CHUNKS)),
            pltpu.SemaphoreType.DMA((2,)),
            pltpu.SemaphoreType.DMA((N_LAYERS, N_CHUNKS)),
            pltpu.SemaphoreType.DMA((N_LAYERS, N_CHUNKS)),
            pltpu.SemaphoreType.DMA((N_LAYERS,)),
            pltpu.SemaphoreType.DMA((N_LAYERS,)),
        ],
        compiler_params=pltpu.CompilerParams(
            collective_id=0,
            vmem_limit_bytes=100 * 1024 * 1024,
        ),
    )(x, Win0, Wout0, Win1, Wout1, Win2, Wout2)
